# baseline (speedup 1.0000x reference)
import functools

import jax
import jax.numpy as jnp
from jax import lax
from jax.experimental import pallas as pl
from jax.experimental.pallas import tpu as pltpu
from jax.experimental.pallas import tpu_sc as plsc

_IDXW = 128
_K = 4


@functools.lru_cache(maxsize=None)
def _build(N, V, D):
    info = plsc.get_sparse_core_info()
    NC, NS = info.num_cores, info.num_subcores
    NW = NC * NS
    n_rows = N // _IDXW
    rows_per_w = n_rows // NW
    n_groups = rows_per_w // _K
    assert N % _IDXW == 0 and n_rows % NW == 0 and rows_per_w % _K == 0

    mesh = plsc.VectorSubcoreMesh(core_axis_name="c", subcore_axis_name="s")

    @functools.partial(
        pl.kernel,
        out_type=jax.ShapeDtypeStruct((n_rows, _IDXW, D), jnp.float32),
        mesh=mesh,
        scratch_types=[
            pltpu.VMEM((_K, _IDXW), jnp.int32),
            pltpu.VMEM((_K, _IDXW, D), jnp.float32),
            pltpu.SemaphoreType.DMA,
        ],
        compiler_params=pltpu.CompilerParams(use_tc_tiling_on_sc=False),
    )
    def k(idx_hbm, table_hbm, out_hbm, idx_v, rows_v, gsem):
        wid = lax.axis_index("s") * NC + lax.axis_index("c")
        row_base = wid * rows_per_w

        def body(g, _):
            base = row_base + g * _K
            pltpu.sync_copy(idx_hbm.at[pl.ds(base, _K)], idx_v)
            cps = [
                pltpu.async_copy(
                    table_hbm.at[idx_v.at[j]], rows_v.at[j], gsem
                )
                for j in range(_K)
            ]
            for cp in cps:
                cp.wait()
            pltpu.sync_copy(rows_v, out_hbm.at[pl.ds(base, _K)])
            return 0

        lax.fori_loop(0, n_groups, body, 0)

    return k


def kernel(X, wv):
    B, H = X.shape
    V, D = wv.shape
    N = B * H
    idx2d = X.reshape(N // _IDXW, _IDXW)
    out = _build(N, V, D)(idx2d, wv)
    return out.reshape(B, H, D)


# speedup vs baseline: 1.4280x; 1.4280x over previous
import functools

import jax
import jax.numpy as jnp
from jax import lax
from jax.experimental import pallas as pl
from jax.experimental.pallas import tpu as pltpu
from jax.experimental.pallas import tpu_sc as plsc

_IDXW = 128
_K = 4
_DP = 128


@functools.lru_cache(maxsize=None)
def _build(N, V, D):
    info = plsc.get_sparse_core_info()
    NC, NS = info.num_cores, info.num_subcores
    NW = NC * NS
    n_rows = N // _IDXW
    rows_per_w = n_rows // NW
    n_groups = rows_per_w // _K
    assert N % _IDXW == 0 and n_rows % NW == 0 and rows_per_w % _K == 0

    mesh = plsc.VectorSubcoreMesh(core_axis_name="c", subcore_axis_name="s")

    @functools.partial(
        pl.kernel,
        out_type=jax.ShapeDtypeStruct((n_rows, _IDXW, _DP), jnp.float32),
        mesh=mesh,
        scratch_types=[
            pltpu.VMEM((_K, _IDXW), jnp.int32),
            pltpu.VMEM((_K, _IDXW, _DP), jnp.float32),
            pltpu.SemaphoreType.DMA,
        ],
    )
    def k(idx_hbm, table_hbm, out_hbm, idx_v, rows_v, gsem):
        wid = lax.axis_index("s") * NC + lax.axis_index("c")
        row_base = wid * rows_per_w

        def body(g, _):
            base = row_base + g * _K
            pltpu.sync_copy(idx_hbm.at[pl.ds(base, _K)], idx_v)
            cps = [
                pltpu.async_copy(
                    table_hbm.at[idx_v.at[j]], rows_v.at[j], gsem
                )
                for j in range(_K)
            ]
            for cp in cps:
                cp.wait()
            pltpu.sync_copy(rows_v, out_hbm.at[pl.ds(base, _K)])
            return 0

        lax.fori_loop(0, n_groups, body, 0)

    return k


def kernel(X, wv):
    B, H = X.shape
    V, D = wv.shape
    N = B * H
    idx2d = X.reshape(N // _IDXW, _IDXW)
    wv_p = jnp.pad(wv, ((0, 0), (0, _DP - D)))
    out = _build(N, V, D)(idx2d, wv_p)
    return out[:, :, :D].reshape(B, H, D)


# speedup vs baseline: 2.6072x; 1.8257x over previous
import functools

import jax
import jax.numpy as jnp
from jax import lax
from jax.experimental import pallas as pl
from jax.experimental.pallas import tpu as pltpu
from jax.experimental.pallas import tpu_sc as plsc

_IDXW = 128
_K = 4
_DP = 128
_PAD_BLK = 4000


def _pad_body(src_ref, dst_ref):
    blk = src_ref[...]
    d = blk.shape[-1]
    dst_ref[:, :d] = blk
    dst_ref[:, d:] = jnp.zeros((blk.shape[0], _DP - d), jnp.float32)


@functools.lru_cache(maxsize=None)
def _build_pad(V, D):
    return pl.pallas_call(
        _pad_body,
        grid=(V // _PAD_BLK,),
        in_specs=[pl.BlockSpec((_PAD_BLK, D), lambda i: (i, 0))],
        out_specs=pl.BlockSpec((_PAD_BLK, _DP), lambda i: (i, 0)),
        out_shape=jax.ShapeDtypeStruct((V, _DP), jnp.float32),
    )


@functools.lru_cache(maxsize=None)
def _build_gather(N, V, D):
    info = plsc.get_sparse_core_info()
    NC, NS = info.num_cores, info.num_subcores
    NW = NC * NS
    n_rows = N // _IDXW
    rows_per_w = n_rows // NW
    n_groups = rows_per_w // _K
    assert N % _IDXW == 0 and n_rows % NW == 0 and rows_per_w % _K == 0

    mesh = plsc.VectorSubcoreMesh(core_axis_name="c", subcore_axis_name="s")

    @functools.partial(
        pl.kernel,
        out_type=jax.ShapeDtypeStruct((n_rows, _IDXW, _DP), jnp.float32),
        mesh=mesh,
        scratch_types=[
            pltpu.VMEM((_K, _IDXW), jnp.int32),
            pltpu.VMEM((_K, _IDXW, _DP), jnp.float32),
            pltpu.SemaphoreType.DMA,
        ],
    )
    def k(idx_hbm, table_hbm, out_hbm, idx_v, rows_v, gsem):
        wid = lax.axis_index("s") * NC + lax.axis_index("c")
        row_base = wid * rows_per_w

        def body(g, _):
            base = row_base + g * _K
            pltpu.sync_copy(idx_hbm.at[pl.ds(base, _K)], idx_v)
            cps = [
                pltpu.async_copy(
                    table_hbm.at[idx_v.at[j]], rows_v.at[j], gsem
                )
                for j in range(_K)
            ]
            for cp in cps:
                cp.wait()
            pltpu.sync_copy(rows_v, out_hbm.at[pl.ds(base, _K)])
            return 0

        lax.fori_loop(0, n_groups, body, 0)

    return k


def kernel(X, wv):
    B, H = X.shape
    V, D = wv.shape
    N = B * H
    idx2d = X.reshape(N // _IDXW, _IDXW)
    wv_p = _build_pad(V, D)(wv)
    out = _build_gather(N, V, D)(idx2d, wv_p)
    return out[:, :, :D].reshape(B, H, D)


# speedup vs baseline: 2.6783x; 1.0273x over previous
import functools

import jax
import jax.numpy as jnp
from jax import lax
from jax.experimental import pallas as pl
from jax.experimental.pallas import tpu as pltpu
from jax.experimental.pallas import tpu_sc as plsc

_IDXW = 128
_K = 4
_DP = 128
_PAD_BLK = 20000


def _pad_body(src_ref, dst_ref):
    blk = src_ref[...]
    d = blk.shape[-1]
    dst_ref[:, :d] = blk


@functools.lru_cache(maxsize=None)
def _build_pad(V, D):
    return pl.pallas_call(
        _pad_body,
        grid=(V // _PAD_BLK,),
        in_specs=[pl.BlockSpec((_PAD_BLK, D), lambda i: (i, 0))],
        out_specs=pl.BlockSpec((_PAD_BLK, _DP), lambda i: (i, 0)),
        out_shape=jax.ShapeDtypeStruct((V, _DP), jnp.float32),
    )


@functools.lru_cache(maxsize=None)
def _build_gather(N, V, D):
    info = plsc.get_sparse_core_info()
    NC, NS = info.num_cores, info.num_subcores
    NW = NC * NS
    n_rows = N // _IDXW
    rows_per_w = n_rows // NW
    n_groups = rows_per_w // _K
    assert N % _IDXW == 0 and n_rows % NW == 0 and rows_per_w % _K == 0

    mesh = plsc.VectorSubcoreMesh(core_axis_name="c", subcore_axis_name="s")

    @functools.partial(
        pl.kernel,
        out_type=jax.ShapeDtypeStruct((n_rows, _IDXW, _DP), jnp.float32),
        mesh=mesh,
        scratch_types=[
            pltpu.VMEM((_K, _IDXW), jnp.int32),
            pltpu.VMEM((_K, _IDXW, _DP), jnp.float32),
            pltpu.SemaphoreType.DMA,
        ],
    )
    def k(idx_hbm, table_hbm, out_hbm, idx_v, rows_v, gsem):
        wid = lax.axis_index("s") * NC + lax.axis_index("c")
        row_base = wid * rows_per_w

        def body(g, _):
            base = row_base + g * _K
            pltpu.sync_copy(idx_hbm.at[pl.ds(base, _K)], idx_v)
            cps = [
                pltpu.async_copy(
                    table_hbm.at[idx_v.at[j]], rows_v.at[j], gsem
                )
                for j in range(_K)
            ]
            for cp in cps:
                cp.wait()
            pltpu.sync_copy(rows_v, out_hbm.at[pl.ds(base, _K)])
            return 0

        lax.fori_loop(0, n_groups, body, 0)

    return k


def kernel(X, wv):
    B, H = X.shape
    V, D = wv.shape
    N = B * H
    idx2d = X.reshape(N // _IDXW, _IDXW)
    wv_p = _build_pad(V, D)(wv)
    out = _build_gather(N, V, D)(idx2d, wv_p)
    return out[:, :, :D].reshape(B, H, D)
